# Initial kernel scaffold; baseline (speedup 1.0000x reference)
import jax
import jax.numpy as jnp
from jax.experimental import pallas as pl


def kernel(coords, feats):
    raise NotImplementedError("write your pallas kernel here")


# speedup vs baseline: 5.8013x; 5.8013x over previous
import functools

import jax
import jax.numpy as jnp
from jax import lax
from jax.experimental import pallas as pl
from jax.experimental.pallas import tpu as pltpu
from jax.experimental.pallas import tpu_sc as plsc

N = 100000
NP = 100352
PER_TILE = NP // 16
NCH = PER_TILE // 128
F = 32
CELLS2 = 64 * 64 * 64
PIECE = 32768
GDUMP = PIECE
CW = 8


def _iota16():
    return lax.iota(jnp.int32, 16)


def _scalar(v):
    return jnp.sum(jnp.where(_iota16() == 0, v, 0))


def _body(xs, ys, zs, fts, zf, fill2, fill3, ones, zi,
          c2o, f2o, c3o, f3o,
          gridbuf, cnt, comm):
    pl.run_scoped(
        functools.partial(_body_inner, xs, ys, zs, fts, zf, fill2, fill3,
                          ones, zi, c2o, f2o, c3o, f3o, gridbuf, cnt, comm),
        pltpu.VMEM((PER_TILE,), jnp.int32),
        pltpu.VMEM((PER_TILE + 144,), jnp.int32),
        pltpu.VMEM((2064,), jnp.int32),
        pltpu.VMEM((2048,), jnp.int32),
        pltpu.VMEM((128,), jnp.int32),
        pltpu.VMEM((128,), jnp.int32),
        pltpu.VMEM((128, F), jnp.float32),
        pltpu.VMEM((128, CW), jnp.int32),
        pltpu.VMEM((128, F), jnp.float32),
        pltpu.VMEM((128, CW), jnp.int32),
        pltpu.VMEM((128,), jnp.int32),
    )


def _body_inner(xs, ys, zs, fts, zf, fill2, fill3, ones, zi,
                c2o, f2o, c3o, f3o,
                gridbuf, cnt, comm,
                klist, plist, occl, cchunk, commst,
                idxb, fbuf, cstage, zv, fillv, onesv):
    core = lax.axis_index("c")
    tid = lax.axis_index("s")
    is0 = core == 0
    tbase = tid * PER_TILE
    it = _iota16()

    pltpu.sync_copy(zf, zv)
    pltpu.sync_copy(ones, onesv)

    s1 = jnp.where(is0, 2, 3)
    sa = jnp.where(is0, 12, 10)
    sb = jnp.where(is0, 6, 5)

    def _phase0(xv, yv, zv3):
        pltpu.sync_copy(xs.at[pl.ds(tbase, PER_TILE)], xv)
        pltpu.sync_copy(ys.at[pl.ds(tbase, PER_TILE)], yv)
        pltpu.sync_copy(zs.at[pl.ds(tbase, PER_TILE)], zv3)

        def p0(i, _):
            x = xv[pl.ds(i * 16, 16)]
            y = yv[pl.ds(i * 16, 16)]
            z = zv3[pl.ds(i * 16, 16)]
            s1v = jnp.full((16,), s1, jnp.int32)
            k = ((lax.shift_right_logical(x, s1v)
                  << jnp.full((16,), sa, jnp.int32))
                 | (lax.shift_right_logical(y, s1v)
                    << jnp.full((16,), sb, jnp.int32))
                 | lax.shift_right_logical(z, s1v))
            klist[pl.ds(i * 16, 16)] = k
            return 0

        lax.fori_loop(0, PER_TILE // 16, p0, 0, unroll=False)

    pl.run_scoped(_phase0,
                  pltpu.VMEM((PER_TILE,), jnp.int32),
                  pltpu.VMEM((PER_TILE,), jnp.int32),
                  pltpu.VMEM((PER_TILE,), jnp.int32))

    nz = jnp.where(is0, 8, 1)
    zstart = tid * jnp.where(is0, CELLS2 // 16, PIECE // 16)

    def p1(i, _):
        pltpu.sync_copy(zi, cnt.at[pl.ds(zstart + i * 2048, 2048)])
        return 0

    lax.fori_loop(0, nz, p1, 0, unroll=False)
    plsc.subcore_barrier()

    def p2(c, _):
        def cp(j, _):
            idxb[pl.ds(j * 16, 16)] = klist[pl.ds(c * 128 + j * 16, 16)]
            return 0
        lax.fori_loop(0, 8, cp, 0, unroll=True)
        pltpu.sync_copy(onesv, cnt.at[idxb], add=True)
        return 0

    lax.fori_loop(0, NCH, p2, 0, unroll=False)
    plsc.subcore_barrier()

    slot_counts = []

    @pl.when(is0)
    def _():
        for j in range(8):
            pltpu.sync_copy(cnt.at[pl.ds(tid * 16384 + j * 2048, 2048)],
                            cchunk)

            def cb(i, acc):
                v = cchunk[pl.ds(i * 16, 16)]
                return acc + jnp.where(v > 0, 1, 0)

            acc = lax.fori_loop(0, 128, cb, jnp.zeros((16,), jnp.int32),
                                unroll=False)
            slot_counts.append((j, jnp.sum(acc)))
        vals = jnp.zeros((16,), jnp.int32)
        for j, s in slot_counts:
            vals = jnp.where(it == j, s, vals)
        plsc.store_scatter(idxb, [it], vals, mask=it < 16)
        pltpu.sync_copy(idxb.at[pl.ds(0, 8)], comm.at[pl.ds(tid * 8, 8)])

    @pl.when(jnp.logical_not(is0))
    def _():
        pltpu.sync_copy(cnt.at[pl.ds(tid * 2048, 2048)], cchunk)
        vals = jnp.zeros((16,), jnp.int32)
        for j in range(8):
            def cb(i, acc):
                v = cchunk[pl.ds(j * 256 + i * 16, 16)]
                return acc + jnp.where(v > 0, 1, 0)

            acc = lax.fori_loop(0, 16, cb, jnp.zeros((16,), jnp.int32),
                                unroll=False)
            vals = jnp.where(it == j, jnp.sum(acc), vals)
        plsc.store_scatter(idxb, [it], vals, mask=it < 16)
        pltpu.sync_copy(idxb.at[pl.ds(0, 8)], comm.at[pl.ds(tid * 8, 8)])

    plsc.subcore_barrier()
    pltpu.sync_copy(comm, commst)

    def tb(i, acc):
        return acc + commst[pl.ds(i * 16, 16)]

    tot = jnp.sum(lax.fori_loop(0, 8, tb, jnp.zeros((16,), jnp.int32),
                                unroll=False))
    ntail = N - 1 - tot

    def emit_fill(cref, fref, fsrc, start, length):
        def f128(c, _):
            pltpu.sync_copy(zv, fref.at[pl.ds(start + c * 128, 128), :])
            pltpu.sync_copy(fsrc, cref.at[pl.ds(start + c * 128, 128), :])
            return 0

        n128 = length // 128
        lax.fori_loop(0, n128, f128, 0, unroll=False)
        rem = length - n128 * 128

        @pl.when((rem > 0) & (length >= 128))
        def _():
            pltpu.sync_copy(zv, fref.at[pl.ds(start + length - 128, 128), :])
            pltpu.sync_copy(fsrc, cref.at[pl.ds(start + length - 128, 128), :])

        @pl.when(length < 128)
        def _():
            def f16(c, _):
                pltpu.sync_copy(zv.at[pl.ds(0, 16), :],
                                fref.at[pl.ds(start + c * 16, 16), :])
                pltpu.sync_copy(fsrc.at[pl.ds(0, 16), :],
                                cref.at[pl.ds(start + c * 16, 16), :])
                return 0

            n16 = length // 16
            lax.fori_loop(0, n16, f16, 0, unroll=False)
            rem16 = length - n16 * 16

            @pl.when((rem16 > 0) & (length >= 16))
            def _():
                pltpu.sync_copy(zv.at[pl.ds(0, 16), :],
                                fref.at[pl.ds(start + length - 16, 16), :])
                pltpu.sync_copy(fsrc.at[pl.ds(0, 16), :],
                                cref.at[pl.ds(start + length - 16, 16), :])

            @pl.when(length < 16)
            def _():
                def f1(c, _):
                    pltpu.sync_copy(zv.at[pl.ds(0, 1), :],
                                    fref.at[pl.ds(start + c, 1), :])
                    pltpu.sync_copy(fsrc.at[pl.ds(0, 1), :],
                                    cref.at[pl.ds(start + c, 1), :])
                    return 0

                lax.fori_loop(0, length, f1, 0, unroll=False)

    fstart = 1 + tot + (ntail * tid) // 16
    fend = 1 + tot + (ntail * (tid + 1)) // 16

    @pl.when(is0)
    def _():
        pltpu.sync_copy(fill2, fillv)

        @pl.when(tid == 0)
        def _():
            pltpu.sync_copy(fillv.at[pl.ds(0, 1), :], c2o.at[pl.ds(0, 1), :])
            pltpu.sync_copy(zv.at[pl.ds(0, 1), :], f2o.at[pl.ds(0, 1), :])
        emit_fill(c2o, f2o, fillv, fstart, fend - fstart)

    @pl.when(jnp.logical_not(is0))
    def _():
        pltpu.sync_copy(fill3, fillv)

        @pl.when(tid == 0)
        def _():
            pltpu.sync_copy(fillv.at[pl.ds(0, 1), :], c3o.at[pl.ds(0, 1), :])
            pltpu.sync_copy(zv.at[pl.ds(0, 1), :], f3o.at[pl.ds(0, 1), :])
        emit_fill(c3o, f3o, fillv, fstart, fend - fstart)

    def write_rows(cref, fref, pbase, sx, sb_, msk, rowbase, mb):
        def stage_c(loff, nrows_j):
            for j in range(nrows_j):
                cells = occl[pl.ds(loff + j * 16, 16)]
                g = cells + pbase
                rows = it + j * 16
                plsc.store_scatter(cstage, [rows, jnp.zeros((16,), jnp.int32)],
                                   lax.shift_right_logical(g, jnp.full((16,), sx, jnp.int32)))
                plsc.store_scatter(cstage, [rows, jnp.ones((16,), jnp.int32)],
                                   lax.shift_right_logical(g, jnp.full((16,), sb_, jnp.int32)) & msk)
                plsc.store_scatter(cstage, [rows, jnp.full((16,), 2, jnp.int32)],
                                   g & msk)

        def w128(loff, orow):
            def cp(j, _):
                idxb[pl.ds(j * 16, 16)] = occl[pl.ds(loff + j * 16, 16)]
                return 0
            lax.fori_loop(0, 8, cp, 0, unroll=True)
            pltpu.sync_copy(gridbuf.at[idxb], fbuf)
            pltpu.sync_copy(fbuf, fref.at[pl.ds(orow, 128), :])
            stage_c(loff, 8)
            pltpu.sync_copy(cstage, cref.at[pl.ds(orow, 128), :])

        def w16(loff, orow):
            def cp(j, _):
                idxb[pl.ds(j * 16, 16)] = occl[pl.ds(loff + j * 16, 16)]
                return 0
            lax.fori_loop(0, 1, cp, 0, unroll=True)
            pltpu.sync_copy(gridbuf.at[idxb.at[pl.ds(0, 16)]],
                            fbuf.at[pl.ds(0, 16), :])
            pltpu.sync_copy(fbuf.at[pl.ds(0, 16), :],
                            fref.at[pl.ds(orow, 16), :])
            stage_c(loff, 1)
            pltpu.sync_copy(cstage.at[pl.ds(0, 16), :],
                            cref.at[pl.ds(orow, 16), :])

        n128 = mb // 128
        lax.fori_loop(0, n128,
                      lambda c, _: (w128(c * 128, rowbase + c * 128), 0)[1],
                      0, unroll=False)
        rem = mb - n128 * 128

        @pl.when((rem > 0) & (mb >= 128))
        def _():
            w128(mb - 128, rowbase + mb - 128)

        @pl.when(mb < 128)
        def _():
            n16 = mb // 16
            lax.fori_loop(0, n16,
                          lambda c, _: (w16(c * 16, rowbase + c * 16), 0)[1],
                          0, unroll=False)
            rem16 = mb - n16 * 16

            @pl.when((rem16 > 0) & (mb >= 16))
            def _():
                w16(mb - 16, rowbase + mb - 16)

            @pl.when(mb < 16)
            def _():
                def w1(r, _):
                    cell = _scalar(occl[pl.ds(r, 16)])
                    pltpu.sync_copy(gridbuf.at[pl.ds(cell, 1), :],
                                    fbuf.at[pl.ds(0, 1), :])
                    pltpu.sync_copy(fbuf.at[pl.ds(0, 1), :],
                                    fref.at[pl.ds(rowbase + r, 1), :])
                    g = cell + pbase
                    row0 = jnp.zeros((16,), jnp.int32)
                    val = jnp.where(
                        it == 0,
                        lax.shift_right_logical(g, sx),
                        jnp.where(it == 1,
                                  lax.shift_right_logical(g, sb_) & msk,
                                  g & msk))
                    plsc.store_scatter(cstage, [row0, it], val, mask=it < 8)
                    pltpu.sync_copy(cstage.at[pl.ds(0, 1), :],
                                    cref.at[pl.ds(rowbase + r, 1), :])
                    return 0

                lax.fori_loop(0, mb, w1, 0, unroll=False)

    for p in range(8):
        active = is0 | (p == 0)

        @pl.when(active)
        def _(p=p):
            def z(i, _):
                pltpu.sync_copy(zv, gridbuf.at[pl.ds(tid * 2049 + i * 128,
                                                     128), :])
                return 0

            lax.fori_loop(0, 16, z, 0, unroll=False)
            pltpu.sync_copy(zv.at[pl.ds(0, 1), :],
                            gridbuf.at[pl.ds(tid * 2049 + 2048, 1), :])

        plsc.subcore_barrier()

        noff = jnp.zeros((), jnp.int32)

        @pl.when(active)
        def _(p=p):
            def bl(i, off):
                k = klist[pl.ds(i * 16, 16)]
                m = lax.shift_right_logical(k, jnp.full((16,), 15, jnp.int32)) == p
                pid = tbase + i * 16 + it
                plsc.store_compressed(plist.at[pl.ds(off, 16)], pid, mask=m)
                return off + jnp.sum(jnp.where(m, 1, 0))

            off = lax.fori_loop(0, PER_TILE // 16, bl,
                                jnp.zeros((), jnp.int32), unroll=False)
            for t in range(8):
                plist[pl.ds(off + t * 16, 16)] = jnp.full((16,), tbase,
                                                          jnp.int32)

            nch = (off + 127) // 128

            def gs(c, _):
                def cp(j, _):
                    pidv = plist[pl.ds(c * 128 + j * 16, 16)]
                    kv = plsc.load_gather(klist, [pidv - tbase])
                    pos = c * 128 + j * 16 + it
                    cell = jnp.where(pos >= off, GDUMP + it, kv & 32767)
                    idxb[pl.ds(j * 16, 16)] = cell
                    return 0
                lax.fori_loop(0, 8, cp, 0, unroll=True)
                pltpu.sync_copy(fts.at[plist.at[pl.ds(c * 128, 128)]], fbuf)
                pltpu.sync_copy(fbuf, gridbuf.at[idxb], add=True)
                return 0

            lax.fori_loop(0, nch, gs, 0, unroll=False)

        plsc.subcore_barrier()

        @pl.when(active)
        def _(p=p):
            slotb = jnp.where(is0, p * 16 + tid, tid * 8)
            bstart = jnp.where(is0, (p * 16 + tid) * 2048, tid * 2048)

            def pre(i, acc):
                s = commst[pl.ds(i * 16, 16)]
                return acc + jnp.where(i * 16 + it < slotb, s, 0)

            rowbase = 1 + jnp.sum(
                lax.fori_loop(0, 8, pre, jnp.zeros((16,), jnp.int32),
                              unroll=False))

            pltpu.sync_copy(cnt.at[pl.ds(bstart, 2048)], cchunk)
            lstart = bstart - jnp.where(is0, p * 32768, 0)

            def oc(i, mb):
                v = cchunk[pl.ds(i * 16, 16)]
                m = v > 0
                cells = lstart + i * 16 + it
                plsc.store_compressed(occl.at[pl.ds(mb, 16)], cells, mask=m)
                return mb + jnp.sum(jnp.where(m, 1, 0))

            mb = lax.fori_loop(0, 128, oc, jnp.zeros((), jnp.int32),
                               unroll=False)

            @pl.when(is0)
            def _():
                write_rows(c2o, f2o, p * 32768, 12, 6, 63, rowbase, mb)

            @pl.when(jnp.logical_not(is0))
            def _():
                write_rows(c3o, f3o, 0, 10, 5, 31, rowbase, mb)

        plsc.subcore_barrier()


@jax.jit
def kernel(coords, feats):
    cpad = jnp.broadcast_to(coords[0], (NP - N, 3))
    cp = jnp.concatenate([coords, cpad], axis=0)
    xs = cp[:, 0]
    ys = cp[:, 1]
    zs = cp[:, 2]
    fts = jnp.concatenate([feats, jnp.zeros((NP - N, F), jnp.float32)],
                          axis=0)
    zf = jnp.zeros((128, F), jnp.float32)
    colpat = jnp.array([-1, 63, 63, 0, 0, 0, 0, 0], jnp.int32)
    fill2 = jnp.broadcast_to(colpat, (128, CW))
    colpat3 = jnp.array([-1, 31, 31, 0, 0, 0, 0, 0], jnp.int32)
    fill3 = jnp.broadcast_to(colpat3, (128, CW))
    ones = jnp.ones((128,), jnp.int32)
    zi = jnp.zeros((2048,), jnp.int32)

    mesh = plsc.VectorSubcoreMesh(core_axis_name="c", subcore_axis_name="s",
                                  num_cores=2, num_subcores=16)
    out = pl.kernel(
        _body,
        out_type=[
            jax.ShapeDtypeStruct((N, CW), jnp.int32),
            jax.ShapeDtypeStruct((N, F), jnp.float32),
            jax.ShapeDtypeStruct((N, CW), jnp.int32),
            jax.ShapeDtypeStruct((N, F), jnp.float32),
        ],
        mesh=mesh,
        compiler_params=pltpu.CompilerParams(use_tc_tiling_on_sc=False,
                                             needs_layout_passes=False),
        scratch_types=[
            pltpu.VMEM_SHARED((PIECE + 16, F), jnp.float32),
            pltpu.VMEM_SHARED((CELLS2,), jnp.int32),
            pltpu.VMEM_SHARED((128,), jnp.int32),
        ],
    )(xs, ys, zs, fts, zf, fill2, fill3, ones, zi)
    c2p, f2, c3p, f3 = out
    return (c2p[:, :3], f2, c3p[:, :3], f3)
